# Initial kernel scaffold; baseline (speedup 1.0000x reference)
import jax
import jax.numpy as jnp
from jax.experimental import pallas as pl


def kernel(queries, memory):
    raise NotImplementedError("write your pallas kernel here")


# speedup vs baseline: 21.7505x; 21.7505x over previous
import functools

import jax
import jax.numpy as jnp
from jax.experimental import pallas as pl

_D = 64
_M = 512
_K = 8
_TAU = 0.1
_BLK = 512


def _body(q_ref, mem_ref, out_ref, smax_ref):
    q = q_ref[...]
    mem = mem_ref[...]

    qn = q * jax.lax.rsqrt(jnp.maximum(jnp.sum(q * q, axis=1, keepdims=True), 1e-24))
    mn = mem * jax.lax.rsqrt(jnp.maximum(jnp.sum(mem * mem, axis=1, keepdims=True), 1e-24))

    sim = jnp.dot(qn, mn.T, preferred_element_type=jnp.float32) * (1.0 / _TAU)

    neg = jnp.float32(-jnp.inf)
    cur = sim
    smax = jnp.max(cur, axis=1, keepdims=True)
    thr = smax
    for _ in range(_K - 1):
        cur = jnp.where(cur >= thr, neg, cur)
        thr = jnp.max(cur, axis=1, keepdims=True)

    e = jnp.where(sim >= thr, jnp.exp(sim - smax), 0.0)
    w = e * (1.0 / jnp.sum(e, axis=1, keepdims=True))

    out_ref[...] = jnp.dot(w, mem, preferred_element_type=jnp.float32)
    smax_ref[...] = smax


@functools.partial(jax.jit, static_argnames=())
def kernel(queries, memory):
    b, n, d = queries.shape
    tokens = b * n
    q2 = queries.reshape(tokens, d)
    grid = (tokens // _BLK,)
    out, smax = pl.pallas_call(
        _body,
        grid=grid,
        in_specs=[
            pl.BlockSpec((_BLK, d), lambda i: (i, 0)),
            pl.BlockSpec((_M, d), lambda i: (0, 0)),
        ],
        out_specs=[
            pl.BlockSpec((_BLK, d), lambda i: (i, 0)),
            pl.BlockSpec((_BLK, 1), lambda i: (i, 0)),
        ],
        out_shape=[
            jax.ShapeDtypeStruct((tokens, d), jnp.float32),
            jax.ShapeDtypeStruct((tokens, 1), jnp.float32),
        ],
    )(q2, memory)
    return out.reshape(b, n, d), smax.reshape(b, n, 1)


# speedup vs baseline: 22.8864x; 1.0522x over previous
import functools

import jax
import jax.numpy as jnp
from jax.experimental import pallas as pl

_D = 64
_M = 512
_K = 8
_TAU = 0.1
_BLK = 512


def _body(q_ref, mem_ref, out_ref, smax_ref):
    q = q_ref[...]
    mem = mem_ref[...]

    qn = q * (jax.lax.rsqrt(jnp.maximum(jnp.sum(q * q, axis=1, keepdims=True), 1e-24))
              * (1.0 / _TAU))
    mn = mem * jax.lax.rsqrt(jnp.maximum(jnp.sum(mem * mem, axis=1, keepdims=True), 1e-24))

    sim = jnp.dot(qn, mn.T, preferred_element_type=jnp.float32)

    neg = jnp.float32(-jnp.inf)
    smax = jnp.max(sim, axis=1, keepdims=True)
    thr = smax
    for _ in range(_K - 1):
        thr = jnp.max(jnp.where(sim < thr, sim, neg), axis=1, keepdims=True)

    e = jnp.where(sim >= thr, jnp.exp(sim - smax), 0.0)
    denom = jnp.sum(e, axis=1, keepdims=True)

    r = jnp.dot(e, mem, preferred_element_type=jnp.float32)
    out_ref[...] = r * (1.0 / denom)
    smax_ref[...] = smax


@functools.partial(jax.jit, static_argnames=())
def kernel(queries, memory):
    b, n, d = queries.shape
    tokens = b * n
    q2 = queries.reshape(tokens, d)
    grid = (tokens // _BLK,)
    out, smax = pl.pallas_call(
        _body,
        grid=grid,
        in_specs=[
            pl.BlockSpec((_BLK, d), lambda i: (i, 0)),
            pl.BlockSpec((_M, d), lambda i: (0, 0)),
        ],
        out_specs=[
            pl.BlockSpec((_BLK, d), lambda i: (i, 0)),
            pl.BlockSpec((_BLK, 1), lambda i: (i, 0)),
        ],
        out_shape=[
            jax.ShapeDtypeStruct((tokens, d), jnp.float32),
            jax.ShapeDtypeStruct((tokens, 1), jnp.float32),
        ],
    )(q2, memory)
    return out.reshape(b, n, d), smax.reshape(b, n, 1)
